# Initial kernel scaffold; baseline (speedup 1.0000x reference)
import jax
import jax.numpy as jnp
from jax.experimental import pallas as pl


def kernel(input_ids, weight, lora_A, lora_B):
    raise NotImplementedError("write your pallas kernel here")


# speedup vs baseline: 3.1271x; 3.1271x over previous
import functools

import jax
import jax.numpy as jnp
from jax import lax
from jax.experimental import pallas as pl
from jax.experimental.pallas import tpu as pltpu
from jax.experimental.pallas import tpu_sc as plsc

D = 64
R = 8
SCALING = 2.0
NC = 2
NS = 16
NW = NC * NS
L = 16

CH = 640
SG = 128


@functools.lru_cache(maxsize=None)
def _build(n_total):
    n_per_w = n_total // NW
    n_chunks = n_per_w // CH
    n_sub = CH // SG

    mesh = plsc.VectorSubcoreMesh(core_axis_name="c", subcore_axis_name="s")

    @functools.partial(
        pl.kernel,
        mesh=mesh,
        out_type=jax.ShapeDtypeStruct((n_total, D), jnp.float32),
        scratch_types=[
            pltpu.VMEM((n_per_w,), jnp.int32),
            pltpu.VMEM((CH, D), jnp.float32),
            pltpu.VMEM((CH, R), jnp.float32),
            pltpu.VMEM((R, D), jnp.float32),
            pltpu.SemaphoreType.DMA,
            pltpu.SemaphoreType.DMA,
        ],
        compiler_params=pltpu.CompilerParams(use_tc_tiling_on_sc=False,
                                             needs_layout_passes=False),
    )
    def k(ids_hbm, w_hbm, a_hbm, b_hbm, out_hbm,
          idx_all, wbuf, bbuf, abuf, semw, semb):
        cid = lax.axis_index("c")
        sid = lax.axis_index("s")
        wid = sid * NC + cid
        base = wid * n_per_w
        pltpu.sync_copy(ids_hbm.at[pl.ds(base, n_per_w)], idx_all)
        pltpu.sync_copy(a_hbm, abuf)

        lane = lax.iota(jnp.int32, L)
        row_of_lane = lane // R
        col_of_lane = lane % R

        def chunk_body(kk, carry):
            cbase = kk * CH
            copies = []
            for j in range(n_sub):
                isl = idx_all.at[pl.ds(cbase + j * SG, SG)]
                copies.append(pltpu.async_copy(
                    w_hbm.at[isl], wbuf.at[pl.ds(j * SG, SG)], semw))
                copies.append(pltpu.async_copy(
                    b_hbm.at[isl], bbuf.at[pl.ds(j * SG, SG)], semb))
            for cp in copies:
                cp.wait()

            def check_body(i, acc):
                rows = i * (L // R) + row_of_lane
                v = plsc.load_gather(bbuf, [rows, col_of_lane])
                return acc | plsc.bitcast(v, jnp.int32)

            nz = lax.fori_loop(0, CH * R // L, check_body,
                               jnp.zeros((L,), jnp.int32))
            any_nz = jnp.any(nz != 0)

            @pl.when(any_nz)
            def _lora_delta():
                def row_body(rr, c2):
                    full_r = jnp.full((L,), rr, jnp.int32)
                    for c in range(D // L):
                        cols = c * L + lane
                        acc = plsc.load_gather(wbuf, [full_r, cols])
                        for r in range(R):
                            bv = plsc.load_gather(
                                bbuf, [full_r, jnp.full((L,), r, jnp.int32)])
                            av = abuf[r, pl.ds(c * L, L)]
                            acc = acc + (bv * SCALING) * av
                        plsc.store_scatter(wbuf, [full_r, cols], acc)
                    return c2

                lax.fori_loop(0, CH, row_body, 0)

            pltpu.sync_copy(wbuf, out_hbm.at[pl.ds(base + cbase, CH)])
            return carry

        lax.fori_loop(0, n_chunks, chunk_body, 0)

    return k


def kernel(input_ids, weight, lora_A, lora_B):
    n_total = input_ids.shape[0] * input_ids.shape[1]
    ids = input_ids.reshape(n_total).astype(jnp.int32)
    out = _build(n_total)(ids, weight, lora_A, lora_B)
    return out.reshape(input_ids.shape + (D,))


# speedup vs baseline: 3.1405x; 1.0043x over previous
import functools

import jax
import jax.numpy as jnp
from jax import lax
from jax.experimental import pallas as pl
from jax.experimental.pallas import tpu as pltpu
from jax.experimental.pallas import tpu_sc as plsc

D = 64
R = 8
SCALING = 2.0
NC = 2
NS = 16
NW = NC * NS
L = 16

SG = 128

_SC_PARAMS = pltpu.CompilerParams(use_tc_tiling_on_sc=False,
                                  needs_layout_passes=False)


@functools.lru_cache(maxsize=None)
def _build_fast(n_total):
    n_per_w = n_total // NW
    CH = 256
    n_chunks = n_per_w // CH
    n_sub = CH // SG

    mesh = plsc.VectorSubcoreMesh(core_axis_name="c", subcore_axis_name="s")

    @functools.partial(
        pl.kernel,
        mesh=mesh,
        out_type=jax.ShapeDtypeStruct((n_total * D,), jnp.float32),
        scratch_types=[
            pltpu.VMEM((n_per_w,), jnp.int32),
            pltpu.VMEM((n_per_w,), jnp.int32),
            pltpu.VMEM((CH, 2 * D), jnp.float32),
            pltpu.VMEM((CH * D,), jnp.float32),
            pltpu.SemaphoreType.DMA,
        ],
        compiler_params=_SC_PARAMS,
    )
    def k(idp_hbm, par_hbm, w2_hbm, out_hbm, idp_v, par_v, pairbuf, obuf, sem):
        cid = lax.axis_index("c")
        sid = lax.axis_index("s")
        wid = sid * NC + cid
        base = wid * n_per_w
        pltpu.sync_copy(idp_hbm.at[pl.ds(base, n_per_w)], idp_v)
        pltpu.sync_copy(par_hbm.at[pl.ds(base, n_per_w)], par_v)

        lane = lax.iota(jnp.int32, L)

        def chunk_body(kk, carry):
            cbase = kk * CH
            copies = []
            for j in range(n_sub):
                isl = idp_v.at[pl.ds(cbase + j * SG, SG)]
                copies.append(pltpu.async_copy(
                    w2_hbm.at[isl], pairbuf.at[pl.ds(j * SG, SG)], sem))
            for cp in copies:
                cp.wait()

            def row_body(rr, c2):
                full_r = jnp.full((L,), rr, jnp.int32)
                pv = plsc.load_gather(
                    par_v, [jnp.full((L,), cbase + rr, jnp.int32)])
                cols0 = pv * D + lane
                for c in range(D // L):
                    v = plsc.load_gather(pairbuf, [full_r, cols0 + c * L])
                    obuf[pl.ds(rr * D + c * L, L)] = v
                return c2

            lax.fori_loop(0, CH, row_body, 0)

            pltpu.sync_copy(obuf, out_hbm.at[pl.ds((base + cbase) * D, CH * D)])
            return carry

        lax.fori_loop(0, n_chunks, chunk_body, 0)

    return k


@functools.lru_cache(maxsize=None)
def _build_slow(n_total):
    n_per_w = n_total // NW
    CH = 640
    n_chunks = n_per_w // CH
    n_sub = CH // SG

    mesh = plsc.VectorSubcoreMesh(core_axis_name="c", subcore_axis_name="s")

    @functools.partial(
        pl.kernel,
        mesh=mesh,
        out_type=jax.ShapeDtypeStruct((n_total, D), jnp.float32),
        scratch_types=[
            pltpu.VMEM((n_per_w,), jnp.int32),
            pltpu.VMEM((CH, D), jnp.float32),
            pltpu.VMEM((CH, R), jnp.float32),
            pltpu.VMEM((R, D), jnp.float32),
            pltpu.SemaphoreType.DMA,
            pltpu.SemaphoreType.DMA,
        ],
        compiler_params=_SC_PARAMS,
    )
    def k(ids_hbm, w_hbm, a_hbm, b_hbm, out_hbm,
          idx_all, wbuf, bbuf, abuf, semw, semb):
        cid = lax.axis_index("c")
        sid = lax.axis_index("s")
        wid = sid * NC + cid
        base = wid * n_per_w
        pltpu.sync_copy(ids_hbm.at[pl.ds(base, n_per_w)], idx_all)
        pltpu.sync_copy(a_hbm, abuf)

        lane = lax.iota(jnp.int32, L)

        def chunk_body(kk, carry):
            cbase = kk * CH
            copies = []
            for j in range(n_sub):
                isl = idx_all.at[pl.ds(cbase + j * SG, SG)]
                copies.append(pltpu.async_copy(
                    w_hbm.at[isl], wbuf.at[pl.ds(j * SG, SG)], semw))
                copies.append(pltpu.async_copy(
                    b_hbm.at[isl], bbuf.at[pl.ds(j * SG, SG)], semb))
            for cp in copies:
                cp.wait()

            def row_body(rr, c2):
                full_r = jnp.full((L,), rr, jnp.int32)
                for c in range(D // L):
                    cols = c * L + lane
                    acc = plsc.load_gather(wbuf, [full_r, cols])
                    for r in range(R):
                        bv = plsc.load_gather(
                            bbuf, [full_r, jnp.full((L,), r, jnp.int32)])
                        av = abuf[r, pl.ds(c * L, L)]
                        acc = acc + (bv * SCALING) * av
                    plsc.store_scatter(wbuf, [full_r, cols], acc)
                return c2

            lax.fori_loop(0, CH, row_body, 0)

            pltpu.sync_copy(wbuf, out_hbm.at[pl.ds(base + cbase, CH)])
            return carry

        lax.fori_loop(0, n_chunks, chunk_body, 0)

    return k


def kernel(input_ids, weight, lora_A, lora_B):
    n_total = input_ids.shape[0] * input_ids.shape[1]
    ids = input_ids.reshape(n_total).astype(jnp.int32)
    w2 = weight.reshape(weight.shape[0] // 2, 2 * D)
    any_nz = jnp.any(lora_B != 0)

    def fast():
        out_flat = _build_fast(n_total)(ids >> 1, ids & 1, w2)
        return out_flat.reshape(n_total, D)

    def slow():
        return _build_slow(n_total)(ids, weight, lora_A, lora_B)

    out = lax.cond(any_nz, slow, fast)
    return out.reshape(input_ids.shape + (D,))


# speedup vs baseline: 3.4854x; 1.1098x over previous
import functools

import jax
import jax.numpy as jnp
from jax import lax
from jax.experimental import pallas as pl
from jax.experimental.pallas import tpu as pltpu
from jax.experimental.pallas import tpu_sc as plsc

D = 64
R = 8
SCALING = 2.0
NC = 2
NS = 16
NW = NC * NS
L = 16

SG = 128

_SC_PARAMS = pltpu.CompilerParams(use_tc_tiling_on_sc=False,
                                  needs_layout_passes=False)


@functools.lru_cache(maxsize=None)
def _build_fast(n_total):
    n_per_w = n_total // NW
    CH = 256
    n_chunks = n_per_w // CH
    n_sub = CH // SG

    mesh = plsc.VectorSubcoreMesh(core_axis_name="c", subcore_axis_name="s")

    @functools.partial(
        pl.kernel,
        mesh=mesh,
        out_type=jax.ShapeDtypeStruct((n_total, D), jnp.float32),
        scratch_types=[
            pltpu.VMEM((n_per_w,), jnp.int32),
            pltpu.VMEM((CH, 2 * D), jnp.float32),
            pltpu.VMEM((CH, D), jnp.float32),
            pltpu.SemaphoreType.DMA,
        ],
        compiler_params=_SC_PARAMS,
    )
    def k(ids_hbm, wp_hbm, out_hbm, idx_v, pairbuf, obuf, sem):
        cid = lax.axis_index("c")
        sid = lax.axis_index("s")
        wid = sid * NC + cid
        base = wid * n_per_w
        pltpu.sync_copy(ids_hbm.at[pl.ds(base, n_per_w)], idx_v)

        lane = lax.iota(jnp.int32, L)

        def chunk_body(kk, carry):
            cbase = kk * CH
            copies = []
            for j in range(n_sub):
                isl = idx_v.at[pl.ds(cbase + j * SG, SG)]
                copies.append(pltpu.async_copy(
                    wp_hbm.at[isl], pairbuf.at[pl.ds(j * SG, SG)], sem))
            for cp in copies:
                cp.wait()

            def row_body(rr, c2):
                full_r = jnp.full((L,), rr, jnp.int32)
                for c in range(D // L):
                    cols = c * L + lane
                    v = plsc.load_gather(pairbuf, [full_r, cols])
                    plsc.store_scatter(obuf, [full_r, cols], v)
                return c2

            lax.fori_loop(0, CH, row_body, 0)

            pltpu.sync_copy(obuf, out_hbm.at[pl.ds(base + cbase, CH)])
            return carry

        lax.fori_loop(0, n_chunks, chunk_body, 0)

    return k


@functools.lru_cache(maxsize=None)
def _build_slow(n_total):
    n_per_w = n_total // NW
    CH = 640
    n_chunks = n_per_w // CH
    n_sub = CH // SG

    mesh = plsc.VectorSubcoreMesh(core_axis_name="c", subcore_axis_name="s")

    @functools.partial(
        pl.kernel,
        mesh=mesh,
        out_type=jax.ShapeDtypeStruct((n_total, D), jnp.float32),
        scratch_types=[
            pltpu.VMEM((n_per_w,), jnp.int32),
            pltpu.VMEM((CH, D), jnp.float32),
            pltpu.VMEM((CH, R), jnp.float32),
            pltpu.VMEM((R, D), jnp.float32),
            pltpu.SemaphoreType.DMA,
            pltpu.SemaphoreType.DMA,
        ],
        compiler_params=_SC_PARAMS,
    )
    def k(ids_hbm, w_hbm, a_hbm, b_hbm, out_hbm,
          idx_all, wbuf, bbuf, abuf, semw, semb):
        cid = lax.axis_index("c")
        sid = lax.axis_index("s")
        wid = sid * NC + cid
        base = wid * n_per_w
        pltpu.sync_copy(ids_hbm.at[pl.ds(base, n_per_w)], idx_all)
        pltpu.sync_copy(a_hbm, abuf)

        lane = lax.iota(jnp.int32, L)

        def chunk_body(kk, carry):
            cbase = kk * CH
            copies = []
            for j in range(n_sub):
                isl = idx_all.at[pl.ds(cbase + j * SG, SG)]
                copies.append(pltpu.async_copy(
                    w_hbm.at[isl], wbuf.at[pl.ds(j * SG, SG)], semw))
                copies.append(pltpu.async_copy(
                    b_hbm.at[isl], bbuf.at[pl.ds(j * SG, SG)], semb))
            for cp in copies:
                cp.wait()

            def row_body(rr, c2):
                full_r = jnp.full((L,), rr, jnp.int32)
                for c in range(D // L):
                    cols = c * L + lane
                    acc = plsc.load_gather(wbuf, [full_r, cols])
                    for r in range(R):
                        bv = plsc.load_gather(
                            bbuf, [full_r, jnp.full((L,), r, jnp.int32)])
                        av = abuf[r, pl.ds(c * L, L)]
                        acc = acc + (bv * SCALING) * av
                    plsc.store_scatter(wbuf, [full_r, cols], acc)
                return c2

            lax.fori_loop(0, CH, row_body, 0)

            pltpu.sync_copy(wbuf, out_hbm.at[pl.ds(base + cbase, CH)])
            return carry

        lax.fori_loop(0, n_chunks, chunk_body, 0)

    return k


def kernel(input_ids, weight, lora_A, lora_B):
    n_total = input_ids.shape[0] * input_ids.shape[1]
    ids = input_ids.reshape(n_total).astype(jnp.int32)
    wp = jnp.pad(weight, ((0, 0), (0, D)))
    any_nz = jnp.any(lora_B != 0)

    def fast():
        return _build_fast(n_total)(ids, wp)

    def slow():
        return _build_slow(n_total)(ids, weight, lora_A, lora_B)

    out = lax.cond(any_nz, slow, fast)
    return out.reshape(input_ids.shape + (D,))
